# baseline (speedup 1.0000x reference)
import functools

import jax
import jax.numpy as jnp
from jax import lax
from jax.experimental import pallas as pl
from jax.experimental.pallas import tpu as pltpu
from jax.experimental.pallas import tpu_sc as plsc

_T = 2048
_D = 768
_H = 64
_E = 64
_BLK = 128
_NB = _T // _BLK
_NCHUNK = 80
_EPG = 8
_NG = _E // _EPG
_HI = jax.lax.Precision.HIGHEST


def _plan_kernel(rin_ref, rl_ref, pos_ref, meta_ref):
    logits = lax.dot_general(rin_ref[...], rl_ref[...], (((1,), (0,)), ((), ())),
                             preferred_element_type=jnp.float32)
    m = jnp.max(logits, axis=1, keepdims=True)
    ii = lax.broadcasted_iota(jnp.int32, (_T, _E), 1)
    idx = jnp.min(jnp.where(logits == m, ii, _E), axis=1, keepdims=True)

    oh = (idx == lax.broadcasted_iota(jnp.int32, (_T, _E), 1)).astype(jnp.float32)
    counts = jnp.sum(oh, axis=0, keepdims=True)
    upper = (lax.broadcasted_iota(jnp.int32, (_E, _E), 0)
             < lax.broadcasted_iota(jnp.int32, (_E, _E), 1)).astype(jnp.float32)
    off = lax.dot_general(counts, upper, (((1,), (0,)), ((), ())),
                          preferred_element_type=jnp.float32, precision=_HI)

    tri = (lax.broadcasted_iota(jnp.int32, (_BLK, _BLK), 1)
           < lax.broadcasted_iota(jnp.int32, (_BLK, _BLK), 0)).astype(jnp.bfloat16)
    run = jnp.zeros((1, _E), jnp.float32)
    for b in range(_NB):
        ohb = oh[b * _BLK:(b + 1) * _BLK, :]
        rank = lax.dot_general(tri, ohb.astype(jnp.bfloat16),
                               (((1,), (0,)), ((), ())),
                               preferred_element_type=jnp.float32)
        posb = jnp.sum(ohb * (off + run + rank), axis=1, keepdims=True)
        pos_ref[b * _BLK:(b + 1) * _BLK, :] = posb.astype(jnp.int32)
        run = run + jnp.sum(ohb, axis=0, keepdims=True)

    inv = jnp.float32(1.0 / _BLK)
    first_blk = jnp.floor(off * inv)
    last_blk = jnp.floor((off + counts - 1.0) * inv)
    nseg = jnp.where(counts > 0, last_blk - first_blk + 1.0, 0.0)
    s = lax.dot_general(nseg, upper, (((1,), (0,)), ((), ())),
                        preferred_element_type=jnp.float32, precision=_HI)

    cc = lax.broadcasted_iota(jnp.int32, (_BLK, _E), 0).astype(jnp.float32)
    ind = (cc >= s) & (cc < s + nseg)
    indf = ind.astype(jnp.float32)
    eef = lax.broadcasted_iota(jnp.int32, (_BLK, _E), 1).astype(jnp.float32)
    eid = jnp.sum(indf * eef, axis=1, keepdims=True)
    sc = jnp.sum(indf * s, axis=1, keepdims=True)
    offc = jnp.sum(indf * off, axis=1, keepdims=True)
    cntc = jnp.sum(indf * counts, axis=1, keepdims=True)
    valid = jnp.sum(indf, axis=1, keepdims=True) > 0.0
    crow = lax.broadcasted_iota(jnp.int32, (_BLK, 1), 0).astype(jnp.float32)
    j = crow - sc
    blk = jnp.floor(offc * inv) + j
    lo = jnp.maximum(offc, blk * _BLK) - blk * _BLK
    hi = jnp.minimum(offc + cntc, (blk + 1.0) * _BLK) - blk * _BLK
    blk = jnp.where(valid, blk, jnp.float32(_NB - 1))
    lo = jnp.where(valid, lo, 0.0)
    hi = jnp.where(valid, hi, 0.0)
    eid = jnp.where(valid, eid, 0.0)
    blk_i = blk.astype(jnp.int32)
    prev = jnp.concatenate([jnp.full((1, 1), -1, jnp.int32), blk_i[:-1, :]], axis=0)
    first = (valid & (blk_i != prev)).astype(jnp.int32)

    eei = lax.broadcasted_iota(jnp.int32, (_BLK, _E), 1)
    crowi = lax.broadcasted_iota(jnp.int32, (_BLK, 1), 0)
    gstart = jnp.sum((eei == crowi * _EPG).astype(jnp.float32) * s,
                     axis=1, keepdims=True)
    k_total = jnp.sum((eei == _E - 1).astype(jnp.float32) * (s + nseg),
                      axis=1, keepdims=True)
    gend = jnp.sum((eei == crowi * _EPG + _EPG).astype(jnp.float32) * s,
                   axis=1, keepdims=True)
    gend = gend + jnp.where(crowi == _NG - 1, k_total, 0.0)
    meta_ref[...] = jnp.concatenate(
        [blk_i, eid.astype(jnp.int32), lo.astype(jnp.int32),
         hi.astype(jnp.int32), first, gstart.astype(jnp.int32),
         gend.astype(jnp.int32), jnp.zeros((_BLK, 1), jnp.int32)], axis=1)


def _ffn_kernel(meta_ref, pes_ref, xs_ref, ge_ref, lin_ref, out_ref, xbf_scr):
    g = pl.program_id(0)

    @pl.when(g == 0)
    def _cast_x():
        xbf_scr[...] = xs_ref[...].astype(jnp.bfloat16)

    base = g * _EPG
    cstart = meta_ref[g, 5]
    cend = meta_ref[g, 6]

    def body(c, carry):
        blk = meta_ref[c, 0]
        eid = meta_ref[c, 1]
        lo = meta_ref[c, 2]
        hi = meta_ref[c, 3]
        first = meta_ref[c, 4]
        el = eid - base
        xb = xbf_scr[pl.ds(blk * _BLK, _BLK), :]
        w = ge_ref[pl.ds(el, 1)].reshape(2 * _H, _D).astype(jnp.bfloat16)
        gg = lax.dot_general(xb, w, (((1,), (1,)), ((), ())),
                             preferred_element_type=jnp.float32)
        act = jax.nn.gelu(gg[:, :_H]) * gg[:, _H:]
        rows = lax.broadcasted_iota(jnp.int32, (_BLK, 1), 0)
        msk = (rows >= lo) & (rows < hi)
        act = act * jnp.where(msk, pes_ref[eid, 0], 0.0)
        lw = lin_ref[pl.ds(el, 1)].reshape(_H, _D).astype(jnp.bfloat16)
        y = lax.dot_general(act.astype(jnp.bfloat16), lw,
                            (((1,), (0,)), ((), ())),
                            preferred_element_type=jnp.float32)
        prevv = out_ref[pl.ds(blk * _BLK, _BLK), :]
        out_ref[pl.ds(blk * _BLK, _BLK), :] = jnp.where(first == 1, y, prevv + y)
        return carry

    lax.fori_loop(cstart, cend, body, 0)


def _sc_scatter_fn(x_hbm, pos_hbm, xs_hbm, pos_v, rows_v, sem):
    nc = 2
    wid = lax.axis_index("s") * nc + lax.axis_index("c")
    rows = _T // 32
    base = wid * rows
    pltpu.sync_copy(pos_hbm.at[pl.ds(base, rows)], pos_v)
    pltpu.sync_copy(x_hbm.at[pl.ds(base, rows)], rows_v)
    pltpu.async_copy(rows_v, xs_hbm.at[pos_v], sem).wait()


def _sc_gather_fn(ys_hbm, pos_hbm, out_hbm, pos_v, rows_v, sem):
    nc = 2
    wid = lax.axis_index("s") * nc + lax.axis_index("c")
    rows = _T // 32
    base = wid * rows
    pltpu.sync_copy(pos_hbm.at[pl.ds(base, rows)], pos_v)
    pltpu.async_copy(ys_hbm.at[pos_v], rows_v, sem).wait()
    pltpu.sync_copy(rows_v, out_hbm.at[pl.ds(base, rows)])


def _sc_call(fn):
    mesh = plsc.VectorSubcoreMesh(core_axis_name="c", subcore_axis_name="s")
    rows = _T // 32
    return functools.partial(
        pl.kernel, mesh=mesh,
        out_type=jax.ShapeDtypeStruct((_T, _D), jnp.float32),
        scratch_types=[
            pltpu.VMEM((rows,), jnp.int32),
            pltpu.VMEM((rows, _D), jnp.float32),
            pltpu.SemaphoreType.DMA,
        ],
    )(fn)


def kernel(x, router_scale, router_logits, gating_einsum, linear, per_expert_scale):
    B, L, D = x.shape
    x32 = x.reshape(L, D).astype(jnp.float32)
    variance = jnp.mean(jnp.square(x32), axis=-1, keepdims=True)
    rin = x32 * lax.rsqrt(variance + 1e-06)
    root = lax.rsqrt(jnp.array(D, dtype=rin.dtype))
    rin = rin * root * router_scale.astype(rin.dtype)

    rin_bf = rin.astype(jnp.bfloat16)
    rl_bf = router_logits.astype(jnp.bfloat16)
    pes2 = per_expert_scale.reshape(_E, 1)

    pos, meta = pl.pallas_call(
        _plan_kernel,
        out_shape=(jax.ShapeDtypeStruct((_T, 1), jnp.int32),
                   jax.ShapeDtypeStruct((_BLK, 8), jnp.int32)),
    )(rin_bf, rl_bf)
    pos1 = pos.reshape(_T)

    xs = _sc_call(_sc_scatter_fn)(x32, pos1)

    ys = pl.pallas_call(
        _ffn_kernel,
        grid=(_NG,),
        in_specs=[
            pl.BlockSpec(memory_space=pltpu.SMEM),
            pl.BlockSpec(memory_space=pltpu.SMEM),
            pl.BlockSpec((_T, _D), lambda g: (0, 0)),
            pl.BlockSpec((_EPG, 2, _H, _D), lambda g: (g, 0, 0, 0)),
            pl.BlockSpec((_EPG, _H, _D), lambda g: (g, 0, 0)),
        ],
        out_specs=pl.BlockSpec((_T, _D), lambda g: (0, 0)),
        out_shape=jax.ShapeDtypeStruct((_T, _D), jnp.float32),
        scratch_shapes=[pltpu.VMEM((_T, _D), jnp.bfloat16)],
    )(meta, pes2, xs, gating_einsum, linear)

    out = _sc_call(_sc_gather_fn)(ys, pos1)
    return out.reshape(B, L, D)
